# baseline (speedup 1.0000x reference)
import functools

import jax
import jax.numpy as jnp
from jax import lax
from jax.experimental import pallas as pl
from jax.experimental.pallas import tpu as pltpu
from jax.experimental.pallas import tpu_sc as plsc

_L = 16
_NC = 2
_NS = 16
_CHUNK = 128
_BLK = 8
_PAIR = 2 * _BLK


def _build_sc_call(n, h, n_chunks_tile, rc, nrc):
    n_pairs_tile = n_chunks_tile // _PAIR
    rows_tile = rc * nrc
    ngrp = h // _L

    mesh = plsc.VectorSubcoreMesh(core_axis_name="c", subcore_axis_name="s")
    out_sds = jax.ShapeDtypeStruct((_NC * n, h), jnp.float32)

    @functools.partial(
        pl.kernel,
        out_type=[out_sds] * 3,
        mesh=mesh,
        compiler_params=pltpu.CompilerParams(use_tc_tiling_on_sc=False),
        scratch_types=[
            pltpu.VMEM((2, _BLK, _CHUNK), jnp.int32),
            pltpu.VMEM((2, _BLK, _CHUNK), jnp.int32),
            pltpu.VMEM((2, _BLK, _CHUNK), jnp.float32),
            pltpu.VMEM((_CHUNK, h), jnp.float32),
            pltpu.VMEM((_CHUNK, h), jnp.float32),
            pltpu.VMEM((_CHUNK, h), jnp.float32),
            pltpu.VMEM((rc, h), jnp.float32),
            pltpu.VMEM((rc, h), jnp.float32),
            pltpu.VMEM_SHARED((n, h), jnp.float32),
            pltpu.SemaphoreType.DMA,
            pltpu.SemaphoreType.DMA,
            pltpu.SemaphoreType.DMA,
            pltpu.SemaphoreType.DMA,
            pltpu.SemaphoreType.DMA,
            pltpu.SemaphoreType.DMA,
            pltpu.SemaphoreType.DMA,
            pltpu.SemaphoreType.DMA,
            pltpu.SemaphoreType.DMA,
            pltpu.SemaphoreType.DMA,
            pltpu.SemaphoreType.DMA,
            pltpu.SemaphoreType.DMA,
        ],
    )
    def sc_call(x0, rows_b, cols_b, vals_b, mean_o, x1_o, x2_o,
                idx_r, idx_c, valb, gath0, gath1, gath2, zbuf, dbuf, acc,
                g0, g1, g2, s0, s1, s2, ir0, ic0, iv0, ir1, ic1, iv1):
        c = lax.axis_index("c")
        s = lax.axis_index("s")
        row_base = s * rows_tile
        hbm_base = c * n + row_base
        gaths = (gath0, gath1, gath2)
        gsems = (g0, g1, g2)
        ssems = (s0, s1, s2)
        isems = ((ir0, ic0, iv0), (ir1, ic1, iv1))

        def idx_copies(chunk_base, which):
            sems = isems[which]
            return (
                pltpu.make_async_copy(
                    rows_b.at[pl.ds(chunk_base, _BLK)],
                    idx_r.at[which], sems[0]),
                pltpu.make_async_copy(
                    cols_b.at[c, pl.ds(chunk_base, _BLK)],
                    idx_c.at[which], sems[1]),
                pltpu.make_async_copy(
                    vals_b.at[pl.ds(chunk_base, _BLK)],
                    valb.at[which], sems[2]),
            )

        def issue_idx(chunk_base, which):
            for d in idx_copies(chunk_base, which):
                d.start()

        def wait_idx(chunk_base, which):
            for d in idx_copies(chunk_base, which):
                d.wait()

        def _z(r, carry):
            for g in range(ngrp):
                zbuf[r, pl.ds(g * _L, _L)] = jnp.zeros((_L,), jnp.float32)
            return carry
        lax.fori_loop(0, rc, _z, None)

        def layer(src, dst):
            def _zero(k, carry):
                pltpu.sync_copy(zbuf, acc.at[pl.ds(row_base + k * rc, rc)])
                return carry
            lax.fori_loop(0, nrc, _zero, None)
            plsc.subcore_barrier()

            issue_idx(s * n_pairs_tile * _PAIR, 0)

            def _pair(p, carry):
                base = (s * n_pairs_tile + p) * _PAIR
                wait_idx(base, 0)
                set1 = idx_copies(base + _BLK, 1)
                for d in set1:
                    d.start()

                gds = [None] * _PAIR
                sds = [None] * _PAIR

                def gather(jj):
                    w, jl = divmod(jj, _BLK)
                    return pltpu.async_copy(
                        src.at[idx_c.at[w, jl]], gaths[jj % 3],
                        gsems[jj % 3])

                gds[0] = gather(0)
                gds[1] = gather(1)
                for jj in range(_PAIR):
                    w, jl = divmod(jj, _BLK)
                    gath = gaths[jj % 3]
                    gds[jj].wait()

                    @plsc.parallel_loop(0, _CHUNK // _L)
                    def _mul(e16):
                        base_e = e16 * _L
                        val16 = valb[w, jl, pl.ds(base_e, _L)]
                        for l in range(_L):
                            vv = jnp.full((_L,), val16[l], jnp.float32)
                            for g in range(ngrp):
                                sl = pl.ds(g * _L, _L)
                                gath[base_e + l, sl] = gath[base_e + l, sl] * vv
                    sds[jj] = pltpu.async_copy(
                        gath, acc.at[idx_r.at[w, jl]], ssems[jj % 3],
                        add=True)
                    if jj == _BLK - 3:
                        for d in set1:
                            d.wait()
                    if jj + 2 < _PAIR:
                        if jj - 1 >= 0:
                            sds[jj - 1].wait()
                        gds[jj + 2] = gather(jj + 2)
                    if jj == _BLK:
                        @pl.when(p + 1 < n_pairs_tile)
                        def _prefetch():
                            issue_idx(base + _PAIR, 0)
                for jj in range(_PAIR - 3, _PAIR):
                    sds[jj].wait()
                return carry
            lax.fori_loop(0, n_pairs_tile, _pair, None)
            plsc.subcore_barrier()

            if dst is not None:
                def _drain(k, carry):
                    pltpu.sync_copy(acc.at[pl.ds(row_base + k * rc, rc)], dbuf)
                    pltpu.sync_copy(dbuf, dst.at[pl.ds(hbm_base + k * rc, rc)])
                    return carry
                lax.fori_loop(0, nrc, _drain, None)

        layer(x0, x1_o)
        layer(x1_o, x2_o)
        layer(x2_o, None)

        def _mean(k, carry):
            sl = pl.ds(hbm_base + k * rc, rc)
            pltpu.sync_copy(x0.at[sl], dbuf)
            for i, xsrc in enumerate((x1_o, x2_o, None)):
                if xsrc is None:
                    pltpu.sync_copy(
                        acc.at[pl.ds(row_base + k * rc, rc)], zbuf)
                else:
                    pltpu.sync_copy(xsrc.at[sl], zbuf)
                scale = 0.25 if i == 2 else 1.0

                def _acc(r, carry2):
                    for g in range(ngrp):
                        ssl = pl.ds(g * _L, _L)
                        dbuf[r, ssl] = (dbuf[r, ssl] + zbuf[r, ssl]) * scale
                    return carry2
                lax.fori_loop(0, rc, _acc, None)
            pltpu.sync_copy(dbuf, mean_o.at[sl])
            return carry
        lax.fori_loop(0, nrc, _mean, None)

    return sc_call


def kernel(user_weight, item_weight, adj_indices, adj_values):
    n_users, d = user_weight.shape
    n_items = item_weight.shape[0]
    n = n_users + n_items
    h = d // 2
    e = adj_values.shape[0]

    per_tile = -(-e // (_NS * _PAIR * _CHUNK)) * (_PAIR * _CHUNK)
    e_pad = per_tile * _NS
    pad = e_pad - e
    rows = jnp.pad(adj_indices[0], (0, pad))
    cols = jnp.pad(adj_indices[1], (0, pad))
    vals = jnp.pad(adj_values, (0, pad))

    rows_tile = n // _NS
    rc = 1
    for cand in range(2, 129):
        if rows_tile % cand == 0:
            rc = cand
    nrc = rows_tile // rc

    all_emb = jnp.concatenate([user_weight, item_weight], axis=0)
    x0 = jnp.concatenate([all_emb[:, :h], all_emb[:, h:]], axis=0)

    rows_b = rows.reshape(-1, _CHUNK)
    cols_b = jnp.stack([cols, cols + n]).reshape(2, -1, _CHUNK)
    vals_b = vals.reshape(-1, _CHUNK)

    sc_call = _build_sc_call(n, h, e_pad // (_NS * _CHUNK), rc, nrc)
    mean_flat, _, _ = sc_call(x0, rows_b, cols_b, vals_b)

    out = jnp.concatenate([mean_flat[:n], mean_flat[n:]], axis=1)
    return out[:n_users], out[n_users:]


# speedup vs baseline: 1.3110x; 1.3110x over previous
import functools

import jax
import jax.numpy as jnp
from jax import lax
from jax.experimental import pallas as pl
from jax.experimental.pallas import tpu as pltpu
from jax.experimental.pallas import tpu_sc as plsc

_L = 16
_NC = 2
_NS = 16
_CHUNK = 128
_BLK = 8


def _build_sc_call(n_users, n_items, d, n_chunks_tile, rc, nrc):
    n = n_users + n_items
    h = d // _NC
    n_blocks_tile = n_chunks_tile // _BLK
    rows_tile = rc * nrc
    ngrp = h // _L

    mesh = plsc.VectorSubcoreMesh(core_axis_name="c", subcore_axis_name="s")

    @functools.partial(
        pl.kernel,
        out_type=[
            jax.ShapeDtypeStruct((n_users, d), jnp.float32),
            jax.ShapeDtypeStruct((n_items, d), jnp.float32),
            jax.ShapeDtypeStruct((_NC * n, h), jnp.float32),
            jax.ShapeDtypeStruct((_NC * n, h), jnp.float32),
        ],
        mesh=mesh,
        compiler_params=pltpu.CompilerParams(use_tc_tiling_on_sc=False),
        scratch_types=[
            pltpu.VMEM((_BLK, _CHUNK), jnp.int32),
            pltpu.VMEM((_BLK, _CHUNK), jnp.int32),
            pltpu.VMEM((_BLK, _CHUNK), jnp.float32),
            pltpu.VMEM((_CHUNK, h), jnp.float32),
            pltpu.VMEM((_CHUNK, h), jnp.float32),
            pltpu.VMEM((_CHUNK, h), jnp.float32),
            pltpu.VMEM((rc, h), jnp.float32),
            pltpu.VMEM((rc, h), jnp.float32),
            pltpu.VMEM_SHARED((n, h), jnp.float32),
            pltpu.SemaphoreType.DMA,
            pltpu.SemaphoreType.DMA,
            pltpu.SemaphoreType.DMA,
            pltpu.SemaphoreType.DMA,
            pltpu.SemaphoreType.DMA,
            pltpu.SemaphoreType.DMA,
        ],
    )
    def sc_call(x0, rows_b, cols_b, vals_b, user_o, item_o, x1_o, x2_o,
                idx_r, idx_c, valb, gath0, gath1, gath2, zbuf, dbuf, acc,
                g0, g1, g2, s0, s1, s2):
        c = lax.axis_index("c")
        s = lax.axis_index("s")
        row_base = s * rows_tile
        hbm_base = c * n + row_base
        col = c * h
        gaths = (gath0, gath1, gath2)
        gsems = (g0, g1, g2)
        ssems = (s0, s1, s2)

        def _z(r, carry):
            for g in range(ngrp):
                zbuf[r, pl.ds(g * _L, _L)] = jnp.zeros((_L,), jnp.float32)
            return carry
        lax.fori_loop(0, rc, _z, None)

        def layer(src, dst):
            def _zero(k, carry):
                pltpu.sync_copy(zbuf, acc.at[pl.ds(row_base + k * rc, rc)])
                return carry
            lax.fori_loop(0, nrc, _zero, None)
            plsc.subcore_barrier()

            def _block(b, carry):
                base = (s * n_blocks_tile + b) * _BLK
                pltpu.sync_copy(rows_b.at[pl.ds(base, _BLK)], idx_r)
                pltpu.sync_copy(cols_b.at[c, pl.ds(base, _BLK)], idx_c)
                pltpu.sync_copy(vals_b.at[pl.ds(base, _BLK)], valb)

                gds = [None] * _BLK
                sds = [None] * _BLK

                def gather(jj):
                    return pltpu.async_copy(
                        src.at[idx_c.at[jj]], gaths[jj % 3], gsems[jj % 3])

                gds[0] = gather(0)
                gds[1] = gather(1)
                for jj in range(_BLK):
                    gath = gaths[jj % 3]
                    gds[jj].wait()

                    @plsc.parallel_loop(0, _CHUNK // _L)
                    def _mul(e16):
                        base_e = e16 * _L
                        val16 = valb[jj, pl.ds(base_e, _L)]
                        for l in range(_L):
                            vv = jnp.full((_L,), val16[l], jnp.float32)
                            for g in range(ngrp):
                                sl = pl.ds(g * _L, _L)
                                gath[base_e + l, sl] = gath[base_e + l, sl] * vv
                    sds[jj] = pltpu.async_copy(
                        gath, acc.at[idx_r.at[jj]], ssems[jj % 3], add=True)
                    if jj + 2 < _BLK:
                        if jj - 1 >= 0:
                            sds[jj - 1].wait()
                        gds[jj + 2] = gather(jj + 2)
                for jj in range(_BLK - 3, _BLK):
                    sds[jj].wait()
                return carry
            lax.fori_loop(0, n_blocks_tile, _block, None)
            plsc.subcore_barrier()

            if dst is not None:
                def _drain(k, carry):
                    pltpu.sync_copy(acc.at[pl.ds(row_base + k * rc, rc)], dbuf)
                    pltpu.sync_copy(dbuf, dst.at[pl.ds(hbm_base + k * rc, rc)])
                    return carry
                lax.fori_loop(0, nrc, _drain, None)

        layer(x0, x1_o)
        layer(x1_o, x2_o)
        layer(x2_o, None)

        def _mean(k, carry):
            row = row_base + k * rc
            sl = pl.ds(hbm_base + k * rc, rc)
            pltpu.sync_copy(x0.at[sl], dbuf)
            for i, xsrc in enumerate((x1_o, x2_o, None)):
                if xsrc is None:
                    pltpu.sync_copy(acc.at[pl.ds(row, rc)], zbuf)
                else:
                    pltpu.sync_copy(xsrc.at[sl], zbuf)
                scale = 0.25 if i == 2 else 1.0

                def _acc(r, carry2):
                    for g in range(ngrp):
                        ssl = pl.ds(g * _L, _L)
                        dbuf[r, ssl] = (dbuf[r, ssl] + zbuf[r, ssl]) * scale
                    return carry2
                lax.fori_loop(0, rc, _acc, None)

            @pl.when(row < n_users)
            def _user():
                pltpu.sync_copy(
                    dbuf, user_o.at[pl.ds(row, rc), pl.ds(col, h)])

            @pl.when(row >= n_users)
            def _item():
                pltpu.sync_copy(
                    dbuf, item_o.at[pl.ds(row - n_users, rc), pl.ds(col, h)])
            return carry
        lax.fori_loop(0, nrc, _mean, None)

    return sc_call


def kernel(user_weight, item_weight, adj_indices, adj_values):
    n_users, d = user_weight.shape
    n_items = item_weight.shape[0]
    n = n_users + n_items
    e = adj_values.shape[0]

    per_tile = -(-e // (_NS * _BLK * _CHUNK)) * (_BLK * _CHUNK)
    e_pad = per_tile * _NS
    pad = e_pad - e
    rows = jnp.pad(adj_indices[0], (0, pad))
    cols = jnp.pad(adj_indices[1], (0, pad))
    vals = jnp.pad(adj_values, (0, pad))

    rows_tile = n // _NS
    rc = 1
    for cand in range(2, 129):
        if rows_tile % cand == 0:
            rc = cand
    nrc = rows_tile // rc

    h = d // _NC
    x0 = jnp.concatenate([user_weight[:, :h], item_weight[:, :h],
                          user_weight[:, h:], item_weight[:, h:]], axis=0)

    rows_b = rows.reshape(-1, _CHUNK)
    cols_b = jnp.stack([cols, cols + n]).reshape(2, -1, _CHUNK)
    vals_b = vals.reshape(-1, _CHUNK)

    sc_call = _build_sc_call(n_users, n_items, d,
                             e_pad // (_NS * _CHUNK), rc, nrc)
    user_emb, item_emb, _, _ = sc_call(x0, rows_b, cols_b, vals_b)
    return user_emb, item_emb


# speedup vs baseline: 1.3172x; 1.0047x over previous
import functools

import jax
import jax.numpy as jnp
from jax import lax
from jax.experimental import pallas as pl
from jax.experimental.pallas import tpu as pltpu
from jax.experimental.pallas import tpu_sc as plsc

_L = 16
_NC = 2
_NS = 16
_CHUNK = 128
_BLK = 8


def _build_sc_call(n_users, n_items, d, n_chunks_tile, rc, nrc):
    n = n_users + n_items
    h = d // _NC
    n_blocks_tile = n_chunks_tile // _BLK
    rows_tile = rc * nrc
    ngrp = h // _L

    mesh = plsc.VectorSubcoreMesh(core_axis_name="c", subcore_axis_name="s")

    @functools.partial(
        pl.kernel,
        out_type=[
            jax.ShapeDtypeStruct((n_users, d), jnp.float32),
            jax.ShapeDtypeStruct((n_items, d), jnp.float32),
            jax.ShapeDtypeStruct((_NC * n, h), jnp.float32),
            jax.ShapeDtypeStruct((_NC * n, h), jnp.float32),
        ],
        mesh=mesh,
        compiler_params=pltpu.CompilerParams(use_tc_tiling_on_sc=False),
        scratch_types=[
            pltpu.VMEM((_BLK, _CHUNK), jnp.int32),
            pltpu.VMEM((_BLK, _CHUNK), jnp.int32),
            pltpu.VMEM((_BLK, _CHUNK), jnp.float32),
            pltpu.VMEM((_CHUNK, h), jnp.float32),
            pltpu.VMEM((_CHUNK, h), jnp.float32),
            pltpu.VMEM((_CHUNK, h), jnp.float32),
            pltpu.VMEM((_CHUNK, h), jnp.float32),
            pltpu.VMEM((_CHUNK, h), jnp.float32),
            pltpu.VMEM((rc, h), jnp.float32),
            pltpu.VMEM_SHARED((n, h), jnp.float32),
            pltpu.SemaphoreType.DMA,
            pltpu.SemaphoreType.DMA,
            pltpu.SemaphoreType.DMA,
            pltpu.SemaphoreType.DMA,
            pltpu.SemaphoreType.DMA,
            pltpu.SemaphoreType.DMA,
        ],
    )
    def sc_call(x0, rows_b, cols_b, vals_b, user_o, item_o, x1_o, x2_o,
                idx_r, idx_c, valb, gath0, gath1, gath2, prod0, prod1,
                dbuf, acc, g0, g1, g2, s0, s1, s2):
        c = lax.axis_index("c")
        s = lax.axis_index("s")
        row_base = s * rows_tile
        hbm_base = c * n + row_base
        col = c * h
        gaths = (gath0, gath1, gath2)
        prods = (prod0, prod1)
        gsems = (g0, g1, g2)
        ssems = (s0, s1, s2)

        def layer(src, dst):
            def _z(r, carry):
                for g in range(ngrp):
                    prod1[r, pl.ds(g * _L, _L)] = jnp.zeros(
                        (_L,), jnp.float32)
                return carry
            lax.fori_loop(0, rc, _z, None)

            def _zero(k, carry):
                pltpu.sync_copy(prod1.at[pl.ds(0, rc)],
                                acc.at[pl.ds(row_base + k * rc, rc)])
                return carry
            lax.fori_loop(0, nrc, _zero, None)
            plsc.subcore_barrier()

            def _block(b, carry):
                base = (s * n_blocks_tile + b) * _BLK
                pltpu.sync_copy(rows_b.at[pl.ds(base, _BLK)], idx_r)
                pltpu.sync_copy(cols_b.at[c, pl.ds(base, _BLK)], idx_c)
                pltpu.sync_copy(vals_b.at[pl.ds(base, _BLK)], valb)

                gds = [None] * _BLK
                sds = [None] * _BLK

                def gather(jj):
                    return pltpu.async_copy(
                        src.at[idx_c.at[jj]], gaths[jj % 3], gsems[jj % 3])

                gds[0] = gather(0)
                gds[1] = gather(1)
                for jj in range(_BLK):
                    gath = gaths[jj % 3]
                    prod = prods[jj % 2]
                    gds[jj].wait()
                    if jj - 2 >= 0:
                        sds[jj - 2].wait()

                    @plsc.parallel_loop(0, _CHUNK // _L)
                    def _mul(e16):
                        base_e = e16 * _L
                        val16 = valb[jj, pl.ds(base_e, _L)]
                        for l in range(_L):
                            vv = jnp.full((_L,), val16[l], jnp.float32)
                            for g in range(ngrp):
                                sl = pl.ds(g * _L, _L)
                                prod[base_e + l, sl] = gath[base_e + l, sl] * vv
                    sds[jj] = pltpu.async_copy(
                        prod, acc.at[idx_r.at[jj]], ssems[jj % 3], add=True)
                    if jj + 2 < _BLK:
                        gds[jj + 2] = gather(jj + 2)
                for jj in range(_BLK - 2, _BLK):
                    sds[jj].wait()
                return carry
            lax.fori_loop(0, n_blocks_tile, _block, None)
            plsc.subcore_barrier()

            if dst is not None:
                def _drain(k, carry):
                    pltpu.sync_copy(acc.at[pl.ds(row_base + k * rc, rc)], dbuf)
                    pltpu.sync_copy(dbuf, dst.at[pl.ds(hbm_base + k * rc, rc)])
                    return carry
                lax.fori_loop(0, nrc, _drain, None)

        layer(x0, x1_o)
        layer(x1_o, x2_o)
        layer(x2_o, None)

        def _mean(k, carry):
            row = row_base + k * rc
            sl = pl.ds(hbm_base + k * rc, rc)
            pltpu.sync_copy(x0.at[sl], dbuf)
            for i, xsrc in enumerate((x1_o, x2_o, None)):
                stage = prod0.at[pl.ds(0, rc)]
                if xsrc is None:
                    pltpu.sync_copy(acc.at[pl.ds(row, rc)], stage)
                else:
                    pltpu.sync_copy(xsrc.at[sl], stage)
                scale = 0.25 if i == 2 else 1.0

                def _acc(r, carry2):
                    for g in range(ngrp):
                        ssl = pl.ds(g * _L, _L)
                        dbuf[r, ssl] = (dbuf[r, ssl] + prod0[r, ssl]) * scale
                    return carry2
                lax.fori_loop(0, rc, _acc, None)

            @pl.when(row < n_users)
            def _user():
                pltpu.sync_copy(
                    dbuf, user_o.at[pl.ds(row, rc), pl.ds(col, h)])

            @pl.when(row >= n_users)
            def _item():
                pltpu.sync_copy(
                    dbuf, item_o.at[pl.ds(row - n_users, rc), pl.ds(col, h)])
            return carry
        lax.fori_loop(0, nrc, _mean, None)

    return sc_call


def kernel(user_weight, item_weight, adj_indices, adj_values):
    n_users, d = user_weight.shape
    n_items = item_weight.shape[0]
    n = n_users + n_items
    e = adj_values.shape[0]

    per_tile = -(-e // (_NS * _BLK * _CHUNK)) * (_BLK * _CHUNK)
    e_pad = per_tile * _NS
    pad = e_pad - e
    rows = jnp.pad(adj_indices[0], (0, pad))
    cols = jnp.pad(adj_indices[1], (0, pad))
    vals = jnp.pad(adj_values, (0, pad))

    rows_tile = n // _NS
    rc = 1
    for cand in range(2, 129):
        if rows_tile % cand == 0:
            rc = cand
    nrc = rows_tile // rc

    h = d // _NC
    x0 = jnp.concatenate([user_weight[:, :h], item_weight[:, :h],
                          user_weight[:, h:], item_weight[:, h:]], axis=0)

    rows_b = rows.reshape(-1, _CHUNK)
    cols_b = jnp.stack([cols, cols + n]).reshape(2, -1, _CHUNK)
    vals_b = vals.reshape(-1, _CHUNK)

    sc_call = _build_sc_call(n_users, n_items, d,
                             e_pad // (_NS * _CHUNK), rc, nrc)
    user_emb, item_emb, _, _ = sc_call(x0, rows_b, cols_b, vals_b)
    return user_emb, item_emb


# speedup vs baseline: 1.3273x; 1.0077x over previous
import functools

import jax
import jax.numpy as jnp
from jax import lax
from jax.experimental import pallas as pl
from jax.experimental.pallas import tpu as pltpu
from jax.experimental.pallas import tpu_sc as plsc

_L = 16
_NC = 2
_NS = 16
_CHUNK = 112
_BLK = 8


def _build_sc_call(n_users, n_items, d, n_chunks_tile, rc, nrc):
    n = n_users + n_items
    h = d // _NC
    n_blocks_tile = n_chunks_tile // _BLK
    rows_tile = rc * nrc
    ngrp = h // _L

    mesh = plsc.VectorSubcoreMesh(core_axis_name="c", subcore_axis_name="s")

    @functools.partial(
        pl.kernel,
        out_type=[
            jax.ShapeDtypeStruct((n_users, d), jnp.float32),
            jax.ShapeDtypeStruct((n_items, d), jnp.float32),
            jax.ShapeDtypeStruct((_NC * n, h), jnp.float32),
            jax.ShapeDtypeStruct((_NC * n, h), jnp.float32),
        ],
        mesh=mesh,
        compiler_params=pltpu.CompilerParams(use_tc_tiling_on_sc=False),
        scratch_types=[
            pltpu.VMEM((_BLK, _CHUNK), jnp.int32),
            pltpu.VMEM((_BLK, _CHUNK), jnp.int32),
            pltpu.VMEM((_BLK, _CHUNK), jnp.float32),
            pltpu.VMEM((_CHUNK, h), jnp.float32),
            pltpu.VMEM((_CHUNK, h), jnp.float32),
            pltpu.VMEM((_CHUNK, h), jnp.float32),
            pltpu.VMEM((_CHUNK, h), jnp.float32),
            pltpu.VMEM((_CHUNK, h), jnp.float32),
            pltpu.VMEM((_CHUNK, h), jnp.float32),
            pltpu.VMEM((rc, h), jnp.float32),
            pltpu.VMEM_SHARED((n, h), jnp.float32),
            pltpu.SemaphoreType.DMA,
            pltpu.SemaphoreType.DMA,
            pltpu.SemaphoreType.DMA,
            pltpu.SemaphoreType.DMA,
            pltpu.SemaphoreType.DMA,
            pltpu.SemaphoreType.DMA,
            pltpu.SemaphoreType.DMA,
        ],
    )
    def sc_call(x0, rows_b, cols_b, vals_b, user_o, item_o, x1_o, x2_o,
                idx_r, idx_c, valb, gath0, gath1, gath2, gath3, prod0,
                prod1, dbuf, acc, g0, g1, g2, g3, s0, s1, s2):
        c = lax.axis_index("c")
        s = lax.axis_index("s")
        row_base = s * rows_tile
        hbm_base = c * n + row_base
        col = c * h
        gaths = (gath0, gath1, gath2, gath3)
        prods = (prod0, prod1)
        gsems = (g0, g1, g2, g3)
        ssems = (s0, s1, s2)

        def layer(src, dst):
            def _z(r, carry):
                for g in range(ngrp):
                    prod1[r, pl.ds(g * _L, _L)] = jnp.zeros(
                        (_L,), jnp.float32)
                return carry
            lax.fori_loop(0, rc, _z, None)

            def _zero(k, carry):
                pltpu.sync_copy(prod1.at[pl.ds(0, rc)],
                                acc.at[pl.ds(row_base + k * rc, rc)])
                return carry
            lax.fori_loop(0, nrc, _zero, None)
            plsc.subcore_barrier()

            def _block(b, carry):
                base = (s * n_blocks_tile + b) * _BLK
                pltpu.sync_copy(rows_b.at[pl.ds(base, _BLK)], idx_r)
                pltpu.sync_copy(cols_b.at[c, pl.ds(base, _BLK)], idx_c)
                pltpu.sync_copy(vals_b.at[pl.ds(base, _BLK)], valb)

                gds = [None] * _BLK
                sds = [None] * _BLK

                def gather(jj):
                    return pltpu.async_copy(
                        src.at[idx_c.at[jj]], gaths[jj % 4], gsems[jj % 4])

                gds[0] = gather(0)
                gds[1] = gather(1)
                gds[2] = gather(2)
                for jj in range(_BLK):
                    gath = gaths[jj % 4]
                    prod = prods[jj % 2]
                    gds[jj].wait()
                    if jj - 2 >= 0:
                        sds[jj - 2].wait()

                    @plsc.parallel_loop(0, _CHUNK // _L)
                    def _mul(e16):
                        base_e = e16 * _L
                        val16 = valb[jj, pl.ds(base_e, _L)]
                        for l in range(_L):
                            vv = jnp.full((_L,), val16[l], jnp.float32)
                            for g in range(ngrp):
                                sl = pl.ds(g * _L, _L)
                                prod[base_e + l, sl] = gath[base_e + l, sl] * vv
                    sds[jj] = pltpu.async_copy(
                        prod, acc.at[idx_r.at[jj]], ssems[jj % 3], add=True)
                    if jj + 3 < _BLK:
                        gds[jj + 3] = gather(jj + 3)
                for jj in range(_BLK - 2, _BLK):
                    sds[jj].wait()
                return carry
            lax.fori_loop(0, n_blocks_tile, _block, None)
            plsc.subcore_barrier()

            if dst is not None:
                def _drain(k, carry):
                    pltpu.sync_copy(acc.at[pl.ds(row_base + k * rc, rc)], dbuf)
                    pltpu.sync_copy(dbuf, dst.at[pl.ds(hbm_base + k * rc, rc)])
                    return carry
                lax.fori_loop(0, nrc, _drain, None)

        layer(x0, x1_o)
        layer(x1_o, x2_o)
        layer(x2_o, None)

        def _mean(k, carry):
            row = row_base + k * rc
            sl = pl.ds(hbm_base + k * rc, rc)
            pltpu.sync_copy(x0.at[sl], dbuf)
            for i, xsrc in enumerate((x1_o, x2_o, None)):
                stage = prod0.at[pl.ds(0, rc)]
                if xsrc is None:
                    pltpu.sync_copy(acc.at[pl.ds(row, rc)], stage)
                else:
                    pltpu.sync_copy(xsrc.at[sl], stage)
                scale = 0.25 if i == 2 else 1.0

                def _acc(r, carry2):
                    for g in range(ngrp):
                        ssl = pl.ds(g * _L, _L)
                        dbuf[r, ssl] = (dbuf[r, ssl] + prod0[r, ssl]) * scale
                    return carry2
                lax.fori_loop(0, rc, _acc, None)

            @pl.when(row < n_users)
            def _user():
                pltpu.sync_copy(
                    dbuf, user_o.at[pl.ds(row, rc), pl.ds(col, h)])

            @pl.when(row >= n_users)
            def _item():
                pltpu.sync_copy(
                    dbuf, item_o.at[pl.ds(row - n_users, rc), pl.ds(col, h)])
            return carry
        lax.fori_loop(0, nrc, _mean, None)

    return sc_call


def kernel(user_weight, item_weight, adj_indices, adj_values):
    n_users, d = user_weight.shape
    n_items = item_weight.shape[0]
    n = n_users + n_items
    e = adj_values.shape[0]

    per_tile = -(-e // (_NS * _BLK * _CHUNK)) * (_BLK * _CHUNK)
    e_pad = per_tile * _NS
    pad = e_pad - e
    rows = jnp.pad(adj_indices[0], (0, pad))
    cols = jnp.pad(adj_indices[1], (0, pad))
    vals = jnp.pad(adj_values, (0, pad))

    rows_tile = n // _NS
    rc = 1
    for cand in range(2, 129):
        if rows_tile % cand == 0:
            rc = cand
    nrc = rows_tile // rc

    h = d // _NC
    x0 = jnp.concatenate([user_weight[:, :h], item_weight[:, :h],
                          user_weight[:, h:], item_weight[:, h:]], axis=0)

    rows_b = rows.reshape(-1, _CHUNK)
    cols_b = jnp.stack([cols, cols + n]).reshape(2, -1, _CHUNK)
    vals_b = vals.reshape(-1, _CHUNK)

    sc_call = _build_sc_call(n_users, n_items, d,
                             e_pad // (_NS * _CHUNK), rc, nrc)
    user_emb, item_emb, _, _ = sc_call(x0, rows_b, cols_b, vals_b)
    return user_emb, item_emb


# speedup vs baseline: 1.3350x; 1.0058x over previous
import functools

import jax
import jax.numpy as jnp
from jax import lax
from jax.experimental import pallas as pl
from jax.experimental.pallas import tpu as pltpu
from jax.experimental.pallas import tpu_sc as plsc

_L = 16
_NC = 2
_NS = 16
_CHUNK = 112
_BLK = 8


def _build_sc_call(n_users, n_items, d, n_chunks_tile, rc, nrc):
    n = n_users + n_items
    h = d // _NC
    n_blocks_tile = n_chunks_tile // _BLK
    rows_tile = rc * nrc
    ngrp = h // _L

    mesh = plsc.VectorSubcoreMesh(core_axis_name="c", subcore_axis_name="s")

    @functools.partial(
        pl.kernel,
        out_type=[
            jax.ShapeDtypeStruct((n_users, d), jnp.float32),
            jax.ShapeDtypeStruct((n_items, d), jnp.float32),
        ],
        mesh=mesh,
        compiler_params=pltpu.CompilerParams(use_tc_tiling_on_sc=False),
        scratch_types=[
            pltpu.HBM((_NC * n, h), jnp.float32),
            pltpu.HBM((_NC * n, h), jnp.float32),
            pltpu.VMEM((_BLK, _CHUNK), jnp.int32),
            pltpu.VMEM((_BLK, _CHUNK), jnp.int32),
            pltpu.VMEM((_BLK, _CHUNK), jnp.float32),
            pltpu.VMEM((_CHUNK, h), jnp.float32),
            pltpu.VMEM((_CHUNK, h), jnp.float32),
            pltpu.VMEM((_CHUNK, h), jnp.float32),
            pltpu.VMEM((_CHUNK, h), jnp.float32),
            pltpu.VMEM((_CHUNK, h), jnp.float32),
            pltpu.VMEM((_CHUNK, h), jnp.float32),
            pltpu.VMEM((rc, h), jnp.float32),
            pltpu.VMEM_SHARED((n, h), jnp.float32),
            pltpu.SemaphoreType.DMA,
            pltpu.SemaphoreType.DMA,
            pltpu.SemaphoreType.DMA,
            pltpu.SemaphoreType.DMA,
            pltpu.SemaphoreType.DMA,
            pltpu.SemaphoreType.DMA,
            pltpu.SemaphoreType.DMA,
        ],
    )
    def sc_call(x0, rows_b, cols_b, vals_b, user_o, item_o,
                x1_o, x2_o,
                idx_r, idx_c, valb, gath0, gath1, gath2, gath3, prod0,
                prod1, dbuf, acc, g0, g1, g2, g3, s0, s1, s2):
        c = lax.axis_index("c")
        s = lax.axis_index("s")
        row_base = s * rows_tile
        hbm_base = c * n + row_base
        col = c * h
        gaths = (gath0, gath1, gath2, gath3)
        prods = (prod0, prod1)
        gsems = (g0, g1, g2, g3)
        ssems = (s0, s1, s2)

        def layer(src, dst):
            def _z(r, carry):
                for g in range(ngrp):
                    prod1[r, pl.ds(g * _L, _L)] = jnp.zeros(
                        (_L,), jnp.float32)
                return carry
            lax.fori_loop(0, rc, _z, None)

            def _zero(k, carry):
                pltpu.sync_copy(prod1.at[pl.ds(0, rc)],
                                acc.at[pl.ds(row_base + k * rc, rc)])
                return carry
            lax.fori_loop(0, nrc, _zero, None)
            plsc.subcore_barrier()

            def _block(b, carry):
                base = (s * n_blocks_tile + b) * _BLK
                pltpu.sync_copy(rows_b.at[pl.ds(base, _BLK)], idx_r)
                pltpu.sync_copy(cols_b.at[c, pl.ds(base, _BLK)], idx_c)
                pltpu.sync_copy(vals_b.at[pl.ds(base, _BLK)], valb)

                gds = [None] * _BLK
                sds = [None] * _BLK

                def gather(jj):
                    return pltpu.async_copy(
                        src.at[idx_c.at[jj]], gaths[jj % 4], gsems[jj % 4])

                gds[0] = gather(0)
                gds[1] = gather(1)
                gds[2] = gather(2)
                for jj in range(_BLK):
                    gath = gaths[jj % 4]
                    prod = prods[jj % 2]
                    gds[jj].wait()
                    if jj - 2 >= 0:
                        sds[jj - 2].wait()

                    @plsc.parallel_loop(0, _CHUNK // _L)
                    def _mul(e16):
                        base_e = e16 * _L
                        val16 = valb[jj, pl.ds(base_e, _L)]
                        for l in range(_L):
                            vv = jnp.full((_L,), val16[l], jnp.float32)
                            for g in range(ngrp):
                                sl = pl.ds(g * _L, _L)
                                prod[base_e + l, sl] = gath[base_e + l, sl] * vv
                    sds[jj] = pltpu.async_copy(
                        prod, acc.at[idx_r.at[jj]], ssems[jj % 3], add=True)
                    if jj + 3 < _BLK:
                        gds[jj + 3] = gather(jj + 3)
                for jj in range(_BLK - 2, _BLK):
                    sds[jj].wait()
                return carry
            lax.fori_loop(0, n_blocks_tile, _block, None)
            plsc.subcore_barrier()

            if dst is not None:
                def _drain(k, carry):
                    pltpu.sync_copy(acc.at[pl.ds(row_base + k * rc, rc)], dbuf)
                    pltpu.sync_copy(dbuf, dst.at[pl.ds(hbm_base + k * rc, rc)])
                    return carry
                lax.fori_loop(0, nrc, _drain, None)

        layer(x0, x1_o)
        layer(x1_o, x2_o)
        layer(x2_o, None)

        def _mean(k, carry):
            row = row_base + k * rc
            sl = pl.ds(hbm_base + k * rc, rc)
            pltpu.sync_copy(x0.at[sl], dbuf)
            for i, xsrc in enumerate((x1_o, x2_o, None)):
                stage = prod0.at[pl.ds(0, rc)]
                if xsrc is None:
                    pltpu.sync_copy(acc.at[pl.ds(row, rc)], stage)
                else:
                    pltpu.sync_copy(xsrc.at[sl], stage)
                scale = 0.25 if i == 2 else 1.0

                def _acc(r, carry2):
                    for g in range(ngrp):
                        ssl = pl.ds(g * _L, _L)
                        dbuf[r, ssl] = (dbuf[r, ssl] + prod0[r, ssl]) * scale
                    return carry2
                lax.fori_loop(0, rc, _acc, None)

            @pl.when(row < n_users)
            def _user():
                pltpu.sync_copy(
                    dbuf, user_o.at[pl.ds(row, rc), pl.ds(col, h)])

            @pl.when(row >= n_users)
            def _item():
                pltpu.sync_copy(
                    dbuf, item_o.at[pl.ds(row - n_users, rc), pl.ds(col, h)])
            return carry
        lax.fori_loop(0, nrc, _mean, None)

    return sc_call


def kernel(user_weight, item_weight, adj_indices, adj_values):
    n_users, d = user_weight.shape
    n_items = item_weight.shape[0]
    n = n_users + n_items
    e = adj_values.shape[0]

    per_tile = -(-e // (_NS * _BLK * _CHUNK)) * (_BLK * _CHUNK)
    e_pad = per_tile * _NS
    pad = e_pad - e
    rows = jnp.pad(adj_indices[0], (0, pad))
    cols = jnp.pad(adj_indices[1], (0, pad))
    vals = jnp.pad(adj_values, (0, pad))

    rows_tile = n // _NS
    rc = 1
    for cand in range(2, 129):
        if rows_tile % cand == 0:
            rc = cand
    nrc = rows_tile // rc

    h = d // _NC
    x0 = jnp.concatenate([user_weight[:, :h], item_weight[:, :h],
                          user_weight[:, h:], item_weight[:, h:]], axis=0)

    rows_b = rows.reshape(-1, _CHUNK)
    cols_b = jnp.stack([cols, cols + n]).reshape(2, -1, _CHUNK)
    vals_b = vals.reshape(-1, _CHUNK)

    sc_call = _build_sc_call(n_users, n_items, d,
                             e_pad // (_NS * _CHUNK), rc, nrc)
    user_emb, item_emb = sc_call(x0, rows_b, cols_b, vals_b)
    return user_emb, item_emb


# speedup vs baseline: 1.3567x; 1.0162x over previous
import functools

import jax
import jax.numpy as jnp
from jax import lax
from jax.experimental import pallas as pl
from jax.experimental.pallas import tpu as pltpu
from jax.experimental.pallas import tpu_sc as plsc

_L = 16
_NC = 2
_NS = 16
_CHUNK = 112
_BLK = 8


def _build_sc_call(n_users, n_items, d, n_chunks_tile, rc, nrc):
    n = n_users + n_items
    h = d // _NC
    n_blocks_tile = n_chunks_tile // _BLK
    rows_tile = rc * nrc
    ngrp = h // _L

    mesh = plsc.VectorSubcoreMesh(core_axis_name="c", subcore_axis_name="s")

    @functools.partial(
        pl.kernel,
        out_type=[
            jax.ShapeDtypeStruct((n_users, d), jnp.float32),
            jax.ShapeDtypeStruct((n_items, d), jnp.float32),
        ],
        mesh=mesh,
        compiler_params=pltpu.CompilerParams(use_tc_tiling_on_sc=False),
        scratch_types=[
            pltpu.HBM((_NC * n, h), jnp.float32),
            pltpu.HBM((_NC * n, h), jnp.float32),
            pltpu.VMEM((_BLK, _CHUNK), jnp.int32),
            pltpu.VMEM((_BLK, _CHUNK), jnp.int32),
            pltpu.VMEM((_BLK, _CHUNK), jnp.float32),
            pltpu.VMEM((_CHUNK, h), jnp.float32),
            pltpu.VMEM((_CHUNK, h), jnp.float32),
            pltpu.VMEM((_CHUNK, h), jnp.float32),
            pltpu.VMEM((_CHUNK, h), jnp.float32),
            pltpu.VMEM((_CHUNK, h), jnp.float32),
            pltpu.VMEM((_CHUNK, h), jnp.float32),
            pltpu.VMEM((rc, h), jnp.float32),
            pltpu.VMEM_SHARED((n, h), jnp.float32),
            pltpu.SemaphoreType.DMA,
            pltpu.SemaphoreType.DMA,
            pltpu.SemaphoreType.DMA,
            pltpu.SemaphoreType.DMA,
            pltpu.SemaphoreType.DMA,
            pltpu.SemaphoreType.DMA,
            pltpu.SemaphoreType.DMA,
        ],
    )
    def sc_call(x0, rows_b, cols_b, vals_b, user_o, item_o,
                x1_o, x2_o,
                idx_r, idx_c, valb, gath0, gath1, gath2, gath3, prod0,
                prod1, dbuf, acc, g0, g1, g2, g3, s0, s1, s2):
        c = lax.axis_index("c")
        s = lax.axis_index("s")
        row_base = s * rows_tile
        hbm_base = c * n + row_base
        col = c * h
        gaths = (gath0, gath1, gath2, gath3)
        prods = (prod0, prod1)
        gsems = (g0, g1, g2, g3)
        ssems = (s0, s1, s2)

        def layer(src, dst):
            def _z(r, carry):
                for g in range(ngrp):
                    prod1[r, pl.ds(g * _L, _L)] = jnp.zeros(
                        (_L,), jnp.float32)
                return carry
            lax.fori_loop(0, rc, _z, None)

            def _zero(k, carry):
                pltpu.sync_copy(prod1.at[pl.ds(0, rc)],
                                acc.at[pl.ds(row_base + k * rc, rc)])
                return carry
            lax.fori_loop(0, nrc, _zero, None)
            plsc.subcore_barrier()

            def _block(b, carry):
                base = (s * n_blocks_tile + b) * _BLK
                pltpu.sync_copy(rows_b.at[pl.ds(base, _BLK)], idx_r)
                pltpu.sync_copy(cols_b.at[pl.ds(base, _BLK)], idx_c)
                cn = jnp.full((_L,), c * n, jnp.int32)

                def _off(j, carry2):
                    for g in range(_CHUNK // _L):
                        sl = pl.ds(g * _L, _L)
                        idx_c[j, sl] = idx_c[j, sl] + cn
                    return carry2
                lax.fori_loop(0, _BLK, _off, None)
                pltpu.sync_copy(vals_b.at[pl.ds(base, _BLK)], valb)

                gds = [None] * _BLK
                sds = [None] * _BLK

                def gather(jj):
                    return pltpu.async_copy(
                        src.at[idx_c.at[jj]], gaths[jj % 4], gsems[jj % 4])

                gds[0] = gather(0)
                gds[1] = gather(1)
                gds[2] = gather(2)
                for jj in range(_BLK):
                    gath = gaths[jj % 4]
                    prod = prods[jj % 2]
                    gds[jj].wait()
                    if jj - 2 >= 0:
                        sds[jj - 2].wait()

                    @plsc.parallel_loop(0, _CHUNK // _L)
                    def _mul(e16):
                        base_e = e16 * _L
                        val16 = valb[jj, pl.ds(base_e, _L)]
                        for l in range(_L):
                            vv = jnp.full((_L,), val16[l], jnp.float32)
                            for g in range(ngrp):
                                sl = pl.ds(g * _L, _L)
                                prod[base_e + l, sl] = gath[base_e + l, sl] * vv
                    sds[jj] = pltpu.async_copy(
                        prod, acc.at[idx_r.at[jj]], ssems[jj % 3], add=True)
                    if jj + 3 < _BLK:
                        gds[jj + 3] = gather(jj + 3)
                for jj in range(_BLK - 2, _BLK):
                    sds[jj].wait()
                return carry
            lax.fori_loop(0, n_blocks_tile, _block, None)
            plsc.subcore_barrier()

            if dst is not None:
                def _drain(k, carry):
                    pltpu.sync_copy(acc.at[pl.ds(row_base + k * rc, rc)], dbuf)
                    pltpu.sync_copy(dbuf, dst.at[pl.ds(hbm_base + k * rc, rc)])
                    return carry
                lax.fori_loop(0, nrc, _drain, None)

        layer(x0, x1_o)
        layer(x1_o, x2_o)
        layer(x2_o, None)

        def _mean(k, carry):
            row = row_base + k * rc
            sl = pl.ds(hbm_base + k * rc, rc)
            pltpu.sync_copy(x0.at[sl], dbuf)
            for i, xsrc in enumerate((x1_o, x2_o, None)):
                stage = prod0.at[pl.ds(0, rc)]
                if xsrc is None:
                    pltpu.sync_copy(acc.at[pl.ds(row, rc)], stage)
                else:
                    pltpu.sync_copy(xsrc.at[sl], stage)
                scale = 0.25 if i == 2 else 1.0

                def _acc(r, carry2):
                    for g in range(ngrp):
                        ssl = pl.ds(g * _L, _L)
                        dbuf[r, ssl] = (dbuf[r, ssl] + prod0[r, ssl]) * scale
                    return carry2
                lax.fori_loop(0, rc, _acc, None)

            @pl.when(row < n_users)
            def _user():
                pltpu.sync_copy(
                    dbuf, user_o.at[pl.ds(row, rc), pl.ds(col, h)])

            @pl.when(row >= n_users)
            def _item():
                pltpu.sync_copy(
                    dbuf, item_o.at[pl.ds(row - n_users, rc), pl.ds(col, h)])
            return carry
        lax.fori_loop(0, nrc, _mean, None)

    return sc_call


def kernel(user_weight, item_weight, adj_indices, adj_values):
    n_users, d = user_weight.shape
    n_items = item_weight.shape[0]
    n = n_users + n_items
    e = adj_values.shape[0]

    per_tile = -(-e // (_NS * _BLK * _CHUNK)) * (_BLK * _CHUNK)
    e_pad = per_tile * _NS
    pad = e_pad - e
    rows = jnp.pad(adj_indices[0], (0, pad))
    cols = jnp.pad(adj_indices[1], (0, pad))
    vals = jnp.pad(adj_values, (0, pad))

    rows_tile = n // _NS
    rc = 1
    for cand in range(2, 129):
        if rows_tile % cand == 0:
            rc = cand
    nrc = rows_tile // rc

    h = d // _NC
    x0 = jnp.concatenate([user_weight[:, :h], item_weight[:, :h],
                          user_weight[:, h:], item_weight[:, h:]], axis=0)

    rows_b = rows.reshape(-1, _CHUNK)
    cols_b = cols.reshape(-1, _CHUNK)
    vals_b = vals.reshape(-1, _CHUNK)

    sc_call = _build_sc_call(n_users, n_items, d,
                             e_pad // (_NS * _CHUNK), rc, nrc)
    user_emb, item_emb = sc_call(x0, rows_b, cols_b, vals_b)
    return user_emb, item_emb
